# baseline (speedup 1.0000x reference)
import functools

import jax
import jax.numpy as jnp
from jax import lax
from jax.experimental import pallas as pl
from jax.experimental.pallas import tpu as pltpu
from jax.experimental.pallas import tpu_sc as plsc

NUM_CORES = 2
NUM_SUBCORES = 16
NUM_WORKERS = NUM_CORES * NUM_SUBCORES
NBUF = 8
SPD = 2
LANES = 16
CH = 2048


def _make_relayout(V, D, L):
    inv_l = float(1.0 / L)
    grid = (V + CH - 1) // CH

    def body(i_ref, o_ref):
        x = i_ref[...] * inv_l
        o_ref[...] = x.reshape(D, CH // 4, 4).transpose(1, 2, 0).reshape(
            CH // 4, 4 * D
        )

    return pl.pallas_call(
        body,
        grid=(grid,),
        in_specs=[pl.BlockSpec((D, CH), lambda i: (0, i))],
        out_specs=pl.BlockSpec((CH // 4, 4 * D), lambda i: (i, 0)),
        out_shape=jax.ShapeDtypeStruct((V // 4, 4 * D), jnp.float32),
    )


def _make_kernel(B, L, D):
    assert B % (NUM_WORKERS * SPD) == 0
    s_per_w = B // NUM_WORKERS
    d_per_w = s_per_w // SPD
    assert d_per_w % NBUF == 0
    rows_per_d = SPD * L
    n_half = D // LANES

    mesh = plsc.VectorSubcoreMesh(core_axis_name="c", subcore_axis_name="s")

    @functools.partial(
        pl.kernel,
        mesh=mesh,
        out_type=jax.ShapeDtypeStruct((B, D), jnp.float32),
        scratch_types=[
            pltpu.VMEM((d_per_w, rows_per_d), jnp.int32),
            pltpu.VMEM((NBUF, rows_per_d, D), jnp.float32),
            pltpu.VMEM((s_per_w, D), jnp.float32),
        ]
        + [pltpu.SemaphoreType.DMA] * NBUF,
        compiler_params=pltpu.CompilerParams(use_tc_tiling_on_sc=False),
    )
    def run(table_hbm, idx_hbm, out_hbm, idx_v, ring_v, out_v, *sems):
        wid = lax.axis_index("s") * NUM_CORES + lax.axis_index("c")
        base = wid * d_per_w

        pltpu.sync_copy(idx_hbm.at[pl.ds(base, d_per_w)], idx_v)

        def gather(d, b):
            return pltpu.make_async_copy(
                table_hbm.at[idx_v.at[d]], ring_v.at[b], sems[b]
            )

        for b in range(NBUF):
            gather(b, b).start()

        def reduce_rows(rows, base_t):
            accs = [
                [rows[base_t, pl.ds(h * LANES, LANES)] for h in range(n_half)],
                [rows[base_t + 1, pl.ds(h * LANES, LANES)] for h in range(n_half)],
            ]
            for t in range(2, L):
                c = accs[t % 2]
                for h in range(n_half):
                    c[h] += rows[base_t + t, pl.ds(h * LANES, LANES)]
            return [accs[0][h] + accs[1][h] for h in range(n_half)]

        def outer(g, _):
            for b in range(NBUF):
                d = g * NBUF + b
                gather(d, b).wait()
                for sp in range(SPD):
                    pooled = reduce_rows(ring_v.at[b], sp * L)
                    s = d * SPD + sp
                    for h in range(n_half):
                        out_v[s, pl.ds(h * LANES, LANES)] = pooled[h]

                @pl.when(d + NBUF < d_per_w)
                def _():
                    gather(d + NBUF, b).start()

            return _

        lax.fori_loop(0, d_per_w // NBUF, outer, None)
        pltpu.sync_copy(out_v, out_hbm.at[pl.ds(wid * s_per_w, s_per_w)])

    return run


def kernel(indices, table):
    B, L = indices.shape
    V, D = table.shape
    idx = indices.astype(jnp.int32).reshape(B // SPD, SPD * L)
    w = _make_relayout(V, D, L)(table.T)
    tbl = w.reshape(V, D)
    return _make_kernel(B, L, D)(tbl, idx)


# speedup vs baseline: 2.9160x; 2.9160x over previous
import functools

import jax
import jax.numpy as jnp
from jax import lax
from jax.experimental import pallas as pl
from jax.experimental.pallas import tpu as pltpu
from jax.experimental.pallas import tpu_sc as plsc

NUM_CORES = 2
NUM_SUBCORES = 16
NUM_WORKERS = NUM_CORES * NUM_SUBCORES
NBUF = 8
SPD = 2
LANES = 16
TNB = 2


def _transpose_slab(slab, outb, rows, row_lo, row_hi):
    for a in range(rows):
        for k in range(8):
            rowv = row_lo if k % 2 == 0 else row_hi
            colv = jnp.full((LANES,), 4 * a + k // 2, jnp.int32)
            outb[a, pl.ds(k * LANES, LANES)] = plsc.load_gather(
                slab, [rowv, colv]
            )


def _make_relayout_sc(V, D):
    assert D == 32
    n_full = V // 128
    tail = V - n_full * 128
    base = n_full // NUM_WORKERS
    extra = n_full - base * NUM_WORKERS

    mesh = plsc.VectorSubcoreMesh(core_axis_name="c", subcore_axis_name="s")

    @functools.partial(
        pl.kernel,
        mesh=mesh,
        out_type=jax.ShapeDtypeStruct((V // 4, 4 * D), jnp.float32),
        scratch_types=[
            pltpu.VMEM((TNB, D, 128), jnp.float32),
            pltpu.VMEM((TNB, 32, 128), jnp.float32),
        ]
        + [pltpu.SemaphoreType.DMA] * (2 * TNB),
        compiler_params=pltpu.CompilerParams(
            use_tc_tiling_on_sc=True, needs_layout_passes=False
        ),
    )
    def run(tt_hbm, tail_hbm, w_hbm, slab_v, outb_v, *sems):
        isems = sems[:TNB]
        osems = sems[TNB:]
        wid = lax.axis_index("s") * NUM_CORES + lax.axis_index("c")
        lo = wid * base + lax.min(wid, extra)
        cnt = base + jnp.where(wid < extra, 1, 0)

        iota = lax.iota(jnp.int32, LANES)
        row_lo = iota
        row_hi = iota + LANES

        def in_copy(cg, b):
            src = tt_hbm.at[:, pl.ds(pl.multiple_of(cg * 128, 128), 128)]
            return pltpu.make_async_copy(src, slab_v.at[b], isems[b])

        def out_copy(cg, b):
            dst = w_hbm.at[pl.ds(pl.multiple_of(cg * 32, 32), 32)]
            return pltpu.make_async_copy(outb_v.at[b], dst, osems[b])

        for b in range(TNB):
            in_copy(lo + b, b).start()

        n_iter = (base + extra + TNB - 1) // TNB

        def body(i, _):
            for b in range(TNB):
                g = i * TNB + b
                cg = lo + g

                @pl.when(g < cnt)
                def _():
                    in_copy(cg, b).wait()

                    @pl.when(g >= TNB)
                    def _():
                        out_copy(cg - TNB, b).wait()

                    _transpose_slab(
                        slab_v.at[b], outb_v.at[b], 32, row_lo, row_hi
                    )
                    out_copy(cg, b).start()

                    @pl.when(g + TNB < cnt)
                    def _():
                        in_copy(cg + TNB, b).start()

            return _

        lax.fori_loop(0, n_iter, body, None)
        for b in range(TNB):
            out_copy(0, b).wait()

        if tail:
            @pl.when(wid == NUM_WORKERS - 1)
            def _():
                pltpu.sync_copy(tail_hbm, outb_v.at[0, pl.ds(0, tail // 4)])
                dst = w_hbm.at[pl.ds(pl.multiple_of(n_full * 32, 32), tail // 4)]
                pltpu.sync_copy(outb_v.at[0, pl.ds(0, tail // 4)], dst)

    return run


def _make_kernel(B, L, D):
    assert B % (NUM_WORKERS * SPD) == 0
    s_per_w = B // NUM_WORKERS
    d_per_w = s_per_w // SPD
    assert d_per_w % NBUF == 0
    rows_per_d = SPD * L
    inv_l = jnp.float32(1.0 / L)
    n_half = D // LANES

    mesh = plsc.VectorSubcoreMesh(core_axis_name="c", subcore_axis_name="s")

    @functools.partial(
        pl.kernel,
        mesh=mesh,
        out_type=jax.ShapeDtypeStruct((B, D), jnp.float32),
        scratch_types=[
            pltpu.VMEM((d_per_w, rows_per_d), jnp.int32),
            pltpu.VMEM((NBUF, rows_per_d, D), jnp.float32),
            pltpu.VMEM((s_per_w, D), jnp.float32),
        ]
        + [pltpu.SemaphoreType.DMA] * NBUF,
        compiler_params=pltpu.CompilerParams(use_tc_tiling_on_sc=False),
    )
    def run(table_hbm, idx_hbm, out_hbm, idx_v, ring_v, out_v, *sems):
        wid = lax.axis_index("s") * NUM_CORES + lax.axis_index("c")
        base = wid * d_per_w

        pltpu.sync_copy(idx_hbm.at[pl.ds(base, d_per_w)], idx_v)

        def gather(d, b):
            return pltpu.make_async_copy(
                table_hbm.at[idx_v.at[d]], ring_v.at[b], sems[b]
            )

        for b in range(NBUF):
            gather(b, b).start()

        def reduce_rows(rows, base_t):
            accs = [
                [rows[base_t, pl.ds(h * LANES, LANES)] for h in range(n_half)],
                [rows[base_t + 1, pl.ds(h * LANES, LANES)] for h in range(n_half)],
            ]
            for t in range(2, L):
                c = accs[t % 2]
                for h in range(n_half):
                    c[h] += rows[base_t + t, pl.ds(h * LANES, LANES)]
            return [(accs[0][h] + accs[1][h]) * inv_l for h in range(n_half)]

        def outer(g, _):
            for b in range(NBUF):
                d = g * NBUF + b
                gather(d, b).wait()
                for sp in range(SPD):
                    pooled = reduce_rows(ring_v.at[b], sp * L)
                    s = d * SPD + sp
                    for h in range(n_half):
                        out_v[s, pl.ds(h * LANES, LANES)] = pooled[h]

                @pl.when(d + NBUF < d_per_w)
                def _():
                    gather(d + NBUF, b).start()

            return _

        lax.fori_loop(0, d_per_w // NBUF, outer, None)
        pltpu.sync_copy(out_v, out_hbm.at[pl.ds(wid * s_per_w, s_per_w)])

    return run


def kernel(indices, table):
    B, L = indices.shape
    V, D = table.shape
    idx = indices.astype(jnp.int32).reshape(B // SPD, SPD * L)
    n_tail = V % 128
    tail16 = table[V - n_tail :, :].reshape(n_tail // 4, 4 * D)
    w = _make_relayout_sc(V, D)(table.T, tail16)
    tbl = w.reshape(V, D)
    return _make_kernel(B, L, D)(tbl, idx)


# speedup vs baseline: 2.9229x; 1.0024x over previous
import functools

import jax
import jax.numpy as jnp
from jax import lax
from jax.experimental import pallas as pl
from jax.experimental.pallas import tpu as pltpu
from jax.experimental.pallas import tpu_sc as plsc

NUM_CORES = 2
NUM_SUBCORES = 16
NUM_WORKERS = NUM_CORES * NUM_SUBCORES
NBUF = 8
SPD = 2
LANES = 16
TNB = 2


def _transpose_slab(slab, outb, rows, row_lo, row_hi):
    for k in range(8):
        rowv = row_lo if k % 2 == 0 else row_hi
        colv = jnp.full((LANES,), k // 2, jnp.int32)
        for a in range(rows):
            outb[a, pl.ds(k * LANES, LANES)] = plsc.load_gather(
                slab, [rowv, colv]
            )
            colv = colv + 4


def _make_relayout_sc(V, D):
    assert D == 32
    n_full = V // 128
    tail = V - n_full * 128
    base = n_full // NUM_WORKERS
    extra = n_full - base * NUM_WORKERS

    mesh = plsc.VectorSubcoreMesh(core_axis_name="c", subcore_axis_name="s")

    @functools.partial(
        pl.kernel,
        mesh=mesh,
        out_type=jax.ShapeDtypeStruct((V // 4, 4 * D), jnp.float32),
        scratch_types=[
            pltpu.VMEM((TNB, D, 128), jnp.float32),
            pltpu.VMEM((TNB, 32, 128), jnp.float32),
        ]
        + [pltpu.SemaphoreType.DMA] * (2 * TNB),
        compiler_params=pltpu.CompilerParams(
            use_tc_tiling_on_sc=True, needs_layout_passes=False
        ),
    )
    def run(tt_hbm, tail_hbm, w_hbm, slab_v, outb_v, *sems):
        isems = sems[:TNB]
        osems = sems[TNB:]
        wid = lax.axis_index("s") * NUM_CORES + lax.axis_index("c")
        lo = wid * base + lax.min(wid, extra)
        cnt = base + jnp.where(wid < extra, 1, 0)

        iota = lax.iota(jnp.int32, LANES)
        row_lo = iota
        row_hi = iota + LANES

        def in_copy(cg, b):
            src = tt_hbm.at[:, pl.ds(pl.multiple_of(cg * 128, 128), 128)]
            return pltpu.make_async_copy(src, slab_v.at[b], isems[b])

        def out_copy(cg, b):
            dst = w_hbm.at[pl.ds(pl.multiple_of(cg * 32, 32), 32)]
            return pltpu.make_async_copy(outb_v.at[b], dst, osems[b])

        for b in range(TNB):
            in_copy(lo + b, b).start()

        n_iter = (base + extra + TNB - 1) // TNB

        def body(i, _):
            for b in range(TNB):
                g = i * TNB + b
                cg = lo + g

                @pl.when(g < cnt)
                def _():
                    in_copy(cg, b).wait()

                    @pl.when(g >= TNB)
                    def _():
                        out_copy(cg - TNB, b).wait()

                    _transpose_slab(
                        slab_v.at[b], outb_v.at[b], 32, row_lo, row_hi
                    )
                    out_copy(cg, b).start()

                    @pl.when(g + TNB < cnt)
                    def _():
                        in_copy(cg + TNB, b).start()

            return _

        lax.fori_loop(0, n_iter, body, None)
        for b in range(TNB):
            out_copy(0, b).wait()

        if tail:
            @pl.when(wid == NUM_WORKERS - 1)
            def _():
                pltpu.sync_copy(tail_hbm, outb_v.at[0, pl.ds(0, tail // 4)])
                dst = w_hbm.at[pl.ds(pl.multiple_of(n_full * 32, 32), tail // 4)]
                pltpu.sync_copy(outb_v.at[0, pl.ds(0, tail // 4)], dst)

    return run


def _make_kernel(B, L, D):
    assert B % (NUM_WORKERS * SPD) == 0
    s_per_w = B // NUM_WORKERS
    d_per_w = s_per_w // SPD
    assert d_per_w % NBUF == 0
    rows_per_d = SPD * L
    inv_l = jnp.float32(1.0 / L)
    n_half = D // LANES

    mesh = plsc.VectorSubcoreMesh(core_axis_name="c", subcore_axis_name="s")

    @functools.partial(
        pl.kernel,
        mesh=mesh,
        out_type=jax.ShapeDtypeStruct((B, D), jnp.float32),
        scratch_types=[
            pltpu.VMEM((d_per_w, rows_per_d), jnp.int32),
            pltpu.VMEM((NBUF, rows_per_d, D), jnp.float32),
            pltpu.VMEM((s_per_w, D), jnp.float32),
        ]
        + [pltpu.SemaphoreType.DMA] * NBUF,
        compiler_params=pltpu.CompilerParams(use_tc_tiling_on_sc=False),
    )
    def run(table_hbm, idx_hbm, out_hbm, idx_v, ring_v, out_v, *sems):
        wid = lax.axis_index("s") * NUM_CORES + lax.axis_index("c")
        base = wid * d_per_w

        pltpu.sync_copy(idx_hbm.at[pl.ds(base, d_per_w)], idx_v)

        def gather(d, b):
            return pltpu.make_async_copy(
                table_hbm.at[idx_v.at[d]], ring_v.at[b], sems[b]
            )

        for b in range(NBUF):
            gather(b, b).start()

        def reduce_rows(rows, base_t):
            accs = [
                [rows[base_t, pl.ds(h * LANES, LANES)] for h in range(n_half)],
                [rows[base_t + 1, pl.ds(h * LANES, LANES)] for h in range(n_half)],
            ]
            for t in range(2, L):
                c = accs[t % 2]
                for h in range(n_half):
                    c[h] += rows[base_t + t, pl.ds(h * LANES, LANES)]
            return [(accs[0][h] + accs[1][h]) * inv_l for h in range(n_half)]

        def outer(g, _):
            for b in range(NBUF):
                d = g * NBUF + b
                gather(d, b).wait()
                for sp in range(SPD):
                    pooled = reduce_rows(ring_v.at[b], sp * L)
                    s = d * SPD + sp
                    for h in range(n_half):
                        out_v[s, pl.ds(h * LANES, LANES)] = pooled[h]

                @pl.when(d + NBUF < d_per_w)
                def _():
                    gather(d + NBUF, b).start()

            return _

        lax.fori_loop(0, d_per_w // NBUF, outer, None)
        pltpu.sync_copy(out_v, out_hbm.at[pl.ds(wid * s_per_w, s_per_w)])

    return run


def kernel(indices, table):
    B, L = indices.shape
    V, D = table.shape
    idx = indices.astype(jnp.int32).reshape(B // SPD, SPD * L)
    n_tail = V % 128
    tail16 = table[V - n_tail :, :].reshape(n_tail // 4, 4 * D)
    w = _make_relayout_sc(V, D)(table.T, tail16)
    tbl = w.reshape(V, D)
    return _make_kernel(B, L, D)(tbl, idx)


# speedup vs baseline: 4.2302x; 1.4472x over previous
import functools

import jax
import jax.numpy as jnp
from jax import lax
from jax.experimental import pallas as pl
from jax.experimental.pallas import tpu as pltpu
from jax.experimental.pallas import tpu_sc as plsc

NUM_CORES = 2
NUM_SUBCORES = 16
NUM_WORKERS = NUM_CORES * NUM_SUBCORES
NBUF = 8
SPD = 2
LANES = 16


def _make_kernel(B, L, D):
    assert B % (NUM_WORKERS * SPD) == 0
    s_per_w = B // NUM_WORKERS
    d_per_w = s_per_w // SPD
    assert d_per_w % NBUF == 0
    rows_per_d = SPD * L
    inv_l = jnp.float32(1.0 / L)
    n_half = D // LANES
    out_rows_w = s_per_w * D // (4 * D)

    mesh = plsc.VectorSubcoreMesh(core_axis_name="c", subcore_axis_name="s")

    @functools.partial(
        pl.kernel,
        mesh=mesh,
        out_type=jax.ShapeDtypeStruct((B * D // (4 * D), 4 * D), jnp.float32),
        scratch_types=[
            pltpu.VMEM((d_per_w, rows_per_d), jnp.int32),
            pltpu.VMEM((NBUF, rows_per_d, D), jnp.float32),
            pltpu.VMEM((out_rows_w, 4 * D), jnp.float32),
        ]
        + [pltpu.SemaphoreType.DMA] * NBUF,
        compiler_params=pltpu.CompilerParams(use_tc_tiling_on_sc=False),
    )
    def run(table_hbm, idx_hbm, out_hbm, idx_v, ring_v, out_v, *sems):
        wid = lax.axis_index("s") * NUM_CORES + lax.axis_index("c")
        base = wid * d_per_w

        pltpu.sync_copy(idx_hbm.at[pl.ds(base, d_per_w)], idx_v)

        def gather(d, b):
            return pltpu.make_async_copy(
                table_hbm.at[idx_v.at[d]], ring_v.at[b], sems[b]
            )

        for b in range(NBUF):
            gather(b, b).start()

        def reduce_rows(rows, base_t):
            accs = [
                [rows[base_t, pl.ds(h * LANES, LANES)] for h in range(n_half)],
                [rows[base_t + 1, pl.ds(h * LANES, LANES)] for h in range(n_half)],
            ]
            for t in range(2, L):
                c = accs[t % 2]
                for h in range(n_half):
                    c[h] += rows[base_t + t, pl.ds(h * LANES, LANES)]
            return [(accs[0][h] + accs[1][h]) * inv_l for h in range(n_half)]

        def outer(g, _):
            for b in range(NBUF):
                d = g * NBUF + b
                gather(d, b).wait()
                for sp in range(SPD):
                    pooled = reduce_rows(ring_v.at[b], sp * L)
                    s = d * SPD + sp
                    for h in range(n_half):
                        off = s * D + h * LANES
                        out_v[off // (4 * D), pl.ds(off % (4 * D), LANES)] = (
                            pooled[h]
                        )

                @pl.when(d + NBUF < d_per_w)
                def _():
                    gather(d + NBUF, b).start()

            return _

        lax.fori_loop(0, d_per_w // NBUF, outer, None)
        pltpu.sync_copy(out_v, out_hbm.at[pl.ds(wid * out_rows_w, out_rows_w)])

    return run


def kernel(indices, table):
    B, L = indices.shape
    V, D = table.shape
    idx = indices.astype(jnp.int32).reshape(B // SPD, SPD * L)
    out4 = _make_kernel(B, L, D)(table, idx)
    return out4.reshape(B, D)
